# baseline (speedup 1.0000x reference)
import functools

import jax
import jax.numpy as jnp
from jax import lax
from jax.experimental import pallas as pl
from jax.experimental.pallas import tpu as pltpu
from jax.experimental.pallas import tpu_sc as plsc

N_NODES = 10000
D_IN = 128
D_HID = 32
N_EDGES = 320000

NCOLS = 48
CNT_COL = D_HID
NC, NS = 2, 16
NW = NC * NS
EPW = N_EDGES // NW
CHUNK = 200
NCHUNK = EPW // CHUNK
ROWS_PER_TILE = N_NODES // NS
ACC_ROWS = N_NODES

ROW_BLK = 10000


def _pre_body(x_ref, wl_ref, wr_ref, yaug_ref, z_ref):
    x = x_ref[...]
    y = jnp.dot(x, wl_ref[...], preferred_element_type=jnp.float32)
    z = jnp.dot(x, wr_ref[...], preferred_element_type=jnp.float32)
    r = y.shape[0]
    ones = jnp.ones((r, 1), jnp.float32)
    zeros = jnp.zeros((r, NCOLS - D_HID - 1), jnp.float32)
    yaug_ref[...] = jnp.concatenate([y, ones, zeros], axis=1)
    z_ref[...] = z


def _pre(x, Wl, Wr):
    grid = (N_NODES // ROW_BLK,)
    return pl.pallas_call(
        _pre_body,
        grid=grid,
        in_specs=[
            pl.BlockSpec((ROW_BLK, D_IN), lambda i: (i, 0)),
            pl.BlockSpec((D_IN, D_HID), lambda i: (0, 0)),
            pl.BlockSpec((D_IN, D_HID), lambda i: (0, 0)),
        ],
        out_specs=[
            pl.BlockSpec((ROW_BLK, NCOLS), lambda i: (i, 0)),
            pl.BlockSpec((ROW_BLK, D_HID), lambda i: (i, 0)),
        ],
        out_shape=[
            jax.ShapeDtypeStruct((N_NODES, NCOLS), jnp.float32),
            jax.ShapeDtypeStruct((N_NODES, D_HID), jnp.float32),
        ],
    )(x, Wl, Wr)


NBUF = 5
NGROUP = NCHUNK // NBUF


def _make_segsum(ncols):
    def body(y_hbm, src2d_hbm, dst2d_hbm, zeros_hbm, out_hbm,
             sidx, didx, *bufs_and_sems):
        rows = bufs_and_sems[:NBUF]
        acc = bufs_and_sems[NBUF]
        gsem = bufs_and_sems[NBUF + 1:2 * NBUF + 1]
        ssem = bufs_and_sems[2 * NBUF + 1:]
        cid = lax.axis_index("c")
        sid = lax.axis_index("s")
        wid = sid * NC + cid

        pltpu.sync_copy(zeros_hbm,
                        acc.at[pl.ds(sid * ROWS_PER_TILE, ROWS_PER_TILE)])
        pltpu.sync_copy(src2d_hbm.at[pl.ds(wid * NCHUNK, NCHUNK)], sidx)
        pltpu.sync_copy(dst2d_hbm.at[pl.ds(wid * NCHUNK, NCHUNK)], didx)
        plsc.subcore_barrier()

        def gather(j, b):
            return pltpu.make_async_copy(y_hbm.at[sidx.at[j]], rows[b], gsem[b])

        def scatter_desc(j, b):
            return pltpu.make_async_copy(rows[b], acc.at[didx.at[j]], ssem[b])

        for b in range(NBUF):
            gather(b, b).start()

        def loop_body(k, carry):
            j0 = k * NBUF
            for b in range(NBUF):
                gather(j0 + b, b).wait()
                pltpu.async_copy(rows[b], acc.at[didx.at[j0 + b]], ssem[b],
                                 add=True)
            for b in range(NBUF):
                scatter_desc(j0 + b, b).wait()
                gather(j0 + NBUF + b, b).start()
            return carry

        lax.fori_loop(0, NGROUP - 1, loop_body, 0)
        j0 = (NGROUP - 1) * NBUF
        for b in range(NBUF):
            gather(j0 + b, b).wait()
            pltpu.async_copy(rows[b], acc.at[didx.at[j0 + b]], ssem[b], add=True)
        for b in range(NBUF):
            scatter_desc(j0 + b, b).wait()
        plsc.subcore_barrier()

        pltpu.sync_copy(acc.at[pl.ds(sid * ROWS_PER_TILE, ROWS_PER_TILE)],
                        out_hbm.at[cid, pl.ds(sid * ROWS_PER_TILE, ROWS_PER_TILE)])

    return functools.partial(
        pl.kernel,
        out_type=jax.ShapeDtypeStruct((NC, ACC_ROWS, ncols), jnp.float32),
        mesh=plsc.VectorSubcoreMesh(core_axis_name="c", subcore_axis_name="s"),
        scratch_types=(
            [pltpu.VMEM((NCHUNK, CHUNK), jnp.int32),
             pltpu.VMEM((NCHUNK, CHUNK), jnp.int32)]
            + [pltpu.VMEM((CHUNK, ncols), jnp.float32) for _ in range(NBUF)]
            + [pltpu.VMEM_SHARED((ACC_ROWS, ncols), jnp.float32)]
            + [pltpu.SemaphoreType.DMA for _ in range(2 * NBUF)]
        ),
        compiler_params=pltpu.CompilerParams(use_tc_tiling_on_sc=False),
    )(body)


_segsum48 = _make_segsum(NCOLS)
_segsum32 = _make_segsum(D_HID)


def _head(acc_feat, cnt, z, b):
    mean = acc_feat / jnp.maximum(cnt, 1.0)
    o = mean + b + z
    norm = jnp.sqrt(jnp.sum(o * o, axis=-1, keepdims=True))
    return o / jnp.maximum(norm, 1e-12)


def _mid_body(p_ref, z_ref, b_ref, wl_ref, wr_ref, y2_ref, z2c_ref):
    acc = p_ref[0] + p_ref[1]
    cnt = acc[:, CNT_COL:CNT_COL + 1]
    h = _head(acc[:, :D_HID], cnt, z_ref[...], b_ref[0:1, :])
    h = jnp.maximum(h, 0.0)
    y2 = jnp.dot(h, wl_ref[...], preferred_element_type=jnp.float32)
    z2 = jnp.dot(h, wr_ref[...], preferred_element_type=jnp.float32)
    r = y2.shape[0]
    zeros = jnp.zeros((r, NCOLS - D_HID - 1), jnp.float32)
    y2_ref[...] = y2
    z2c_ref[...] = jnp.concatenate([z2, cnt, zeros], axis=1)


def _mid(p, z1, b1, W2l, W2r):
    grid = (N_NODES // ROW_BLK,)
    return pl.pallas_call(
        _mid_body,
        grid=grid,
        in_specs=[
            pl.BlockSpec((NC, ROW_BLK, NCOLS), lambda i: (0, i, 0)),
            pl.BlockSpec((ROW_BLK, D_HID), lambda i: (i, 0)),
            pl.BlockSpec((8, D_HID), lambda i: (0, 0)),
            pl.BlockSpec((D_HID, D_HID), lambda i: (0, 0)),
            pl.BlockSpec((D_HID, D_HID), lambda i: (0, 0)),
        ],
        out_specs=[
            pl.BlockSpec((ROW_BLK, D_HID), lambda i: (i, 0)),
            pl.BlockSpec((ROW_BLK, NCOLS), lambda i: (i, 0)),
        ],
        out_shape=[
            jax.ShapeDtypeStruct((N_NODES, D_HID), jnp.float32),
            jax.ShapeDtypeStruct((N_NODES, NCOLS), jnp.float32),
        ],
    )(p, z1, b1, W2l, W2r)


def _post_body(q_ref, z2c_ref, b_ref, out_ref):
    acc = q_ref[0] + q_ref[1]
    z2c = z2c_ref[...]
    cnt = z2c[:, D_HID:D_HID + 1]
    out_ref[...] = _head(acc, cnt, z2c[:, :D_HID], b_ref[0:1, :])


def _post(q, z2c, b2):
    grid = (N_NODES // ROW_BLK,)
    return pl.pallas_call(
        _post_body,
        grid=grid,
        in_specs=[
            pl.BlockSpec((NC, ROW_BLK, D_HID), lambda i: (0, i, 0)),
            pl.BlockSpec((ROW_BLK, NCOLS), lambda i: (i, 0)),
            pl.BlockSpec((8, D_HID), lambda i: (0, 0)),
        ],
        out_specs=pl.BlockSpec((ROW_BLK, D_HID), lambda i: (i, 0)),
        out_shape=jax.ShapeDtypeStruct((N_NODES, D_HID), jnp.float32),
    )(q, z2c, b2)


def kernel(x, edge_index, W1l, b1, W1r, W2l, b2, W2r):
    src2d = edge_index[0].astype(jnp.int32).reshape(NW * NCHUNK, CHUNK)
    dst2d = edge_index[1].astype(jnp.int32).reshape(NW * NCHUNK, CHUNK)
    zeros48 = jnp.zeros((ROWS_PER_TILE, NCOLS), jnp.float32)
    zeros32 = jnp.zeros((ROWS_PER_TILE, D_HID), jnp.float32)
    b1v = jnp.broadcast_to(b1[None, :], (8, D_HID))
    b2v = jnp.broadcast_to(b2[None, :], (8, D_HID))

    yaug1, z1 = _pre(x, W1l, W1r)
    p = _segsum48(yaug1, src2d, dst2d, zeros48)
    y2, z2c = _mid(p, z1, b1v, W2l, W2r)
    q = _segsum32(y2, src2d, dst2d, zeros32)
    return _post(q, z2c, b2v)


# speedup vs baseline: 1.0610x; 1.0610x over previous
import functools

import jax
import jax.numpy as jnp
from jax import lax
from jax.experimental import pallas as pl
from jax.experimental.pallas import tpu as pltpu
from jax.experimental.pallas import tpu_sc as plsc

N_NODES = 10000
D_IN = 128
D_HID = 32
N_EDGES = 320000

NCOLS = 40
CNT_COL = D_HID
NC, NS = 2, 16
NW = NC * NS
EPW = N_EDGES // NW
CHUNK = 200
NCHUNK = EPW // CHUNK
ROWS_PER_TILE = N_NODES // NS
ACC_ROWS = N_NODES

ROW_BLK = 5000


def _pre_body(x_ref, wl_ref, wr_ref, yaug_ref, z_ref):
    x = x_ref[...]
    y = jnp.dot(x, wl_ref[...], preferred_element_type=jnp.float32)
    z = jnp.dot(x, wr_ref[...], preferred_element_type=jnp.float32)
    r = y.shape[0]
    ones = jnp.ones((r, 1), jnp.float32)
    zeros = jnp.zeros((r, NCOLS - D_HID - 1), jnp.float32)
    yaug_ref[...] = jnp.concatenate([y, ones, zeros], axis=1)
    z_ref[...] = z


def _pre(x, Wl, Wr):
    grid = (N_NODES // ROW_BLK,)
    return pl.pallas_call(
        _pre_body,
        grid=grid,
        in_specs=[
            pl.BlockSpec((ROW_BLK, D_IN), lambda i: (i, 0)),
            pl.BlockSpec((D_IN, D_HID), lambda i: (0, 0)),
            pl.BlockSpec((D_IN, D_HID), lambda i: (0, 0)),
        ],
        out_specs=[
            pl.BlockSpec((ROW_BLK, NCOLS), lambda i: (i, 0)),
            pl.BlockSpec((ROW_BLK, D_HID), lambda i: (i, 0)),
        ],
        out_shape=[
            jax.ShapeDtypeStruct((N_NODES, NCOLS), jnp.float32),
            jax.ShapeDtypeStruct((N_NODES, D_HID), jnp.float32),
        ],
    )(x, Wl, Wr)


NBUF = 5
NGROUP = NCHUNK // NBUF


def _make_segsum(ncols):
    def body(y_hbm, src2d_hbm, dst2d_hbm, zeros_hbm, out_hbm,
             sidx, didx, *bufs_and_sems):
        rows = bufs_and_sems[:NBUF]
        acc = bufs_and_sems[NBUF]
        gsem = bufs_and_sems[NBUF + 1:2 * NBUF + 1]
        ssem = bufs_and_sems[2 * NBUF + 1:]
        cid = lax.axis_index("c")
        sid = lax.axis_index("s")
        wid = sid * NC + cid

        pltpu.sync_copy(zeros_hbm,
                        acc.at[pl.ds(sid * ROWS_PER_TILE, ROWS_PER_TILE)])
        pltpu.sync_copy(src2d_hbm.at[pl.ds(wid * NCHUNK, NCHUNK)], sidx)
        pltpu.sync_copy(dst2d_hbm.at[pl.ds(wid * NCHUNK, NCHUNK)], didx)
        plsc.subcore_barrier()

        def gather(j, b):
            return pltpu.make_async_copy(y_hbm.at[sidx.at[j]], rows[b], gsem[b])

        def scatter_desc(j, b):
            return pltpu.make_async_copy(rows[b], acc.at[didx.at[j]], ssem[b])

        for b in range(NBUF):
            gather(b, b).start()

        def loop_body(k, carry):
            j0 = k * NBUF
            for b in range(NBUF):
                gather(j0 + b, b).wait()
                pltpu.async_copy(rows[b], acc.at[didx.at[j0 + b]], ssem[b],
                                 add=True)
            for b in range(NBUF):
                scatter_desc(j0 + b, b).wait()
                gather(j0 + NBUF + b, b).start()
            return carry

        lax.fori_loop(0, NGROUP - 1, loop_body, 0)
        j0 = (NGROUP - 1) * NBUF
        for b in range(NBUF):
            gather(j0 + b, b).wait()
            pltpu.async_copy(rows[b], acc.at[didx.at[j0 + b]], ssem[b], add=True)
        for b in range(NBUF):
            scatter_desc(j0 + b, b).wait()
        plsc.subcore_barrier()

        pltpu.sync_copy(acc.at[pl.ds(sid * ROWS_PER_TILE, ROWS_PER_TILE)],
                        out_hbm.at[cid, pl.ds(sid * ROWS_PER_TILE, ROWS_PER_TILE)])

    return functools.partial(
        pl.kernel,
        out_type=jax.ShapeDtypeStruct((NC, ACC_ROWS, ncols), jnp.float32),
        mesh=plsc.VectorSubcoreMesh(core_axis_name="c", subcore_axis_name="s"),
        scratch_types=(
            [pltpu.VMEM((NCHUNK, CHUNK), jnp.int32),
             pltpu.VMEM((NCHUNK, CHUNK), jnp.int32)]
            + [pltpu.VMEM((CHUNK, ncols), jnp.float32) for _ in range(NBUF)]
            + [pltpu.VMEM_SHARED((ACC_ROWS, ncols), jnp.float32)]
            + [pltpu.SemaphoreType.DMA for _ in range(2 * NBUF)]
        ),
        compiler_params=pltpu.CompilerParams(use_tc_tiling_on_sc=False),
    )(body)


_segsum48 = _make_segsum(NCOLS)
_segsum32 = _make_segsum(D_HID)


def _head(acc_feat, cnt, z, b):
    mean = acc_feat / jnp.maximum(cnt, 1.0)
    o = mean + b + z
    norm = jnp.sqrt(jnp.sum(o * o, axis=-1, keepdims=True))
    return o / jnp.maximum(norm, 1e-12)


def _mid_body(p_ref, z_ref, b_ref, wl_ref, wr_ref, y2_ref, z2c_ref):
    acc = p_ref[0] + p_ref[1]
    cnt = acc[:, CNT_COL:CNT_COL + 1]
    h = _head(acc[:, :D_HID], cnt, z_ref[...], b_ref[0:1, :])
    h = jnp.maximum(h, 0.0)
    y2 = jnp.dot(h, wl_ref[...], preferred_element_type=jnp.float32)
    z2 = jnp.dot(h, wr_ref[...], preferred_element_type=jnp.float32)
    r = y2.shape[0]
    zeros = jnp.zeros((r, NCOLS - D_HID - 1), jnp.float32)
    y2_ref[...] = y2
    z2c_ref[...] = jnp.concatenate([z2, cnt, zeros], axis=1)


def _mid(p, z1, b1, W2l, W2r):
    grid = (N_NODES // ROW_BLK,)
    return pl.pallas_call(
        _mid_body,
        grid=grid,
        in_specs=[
            pl.BlockSpec((NC, ROW_BLK, NCOLS), lambda i: (0, i, 0)),
            pl.BlockSpec((ROW_BLK, D_HID), lambda i: (i, 0)),
            pl.BlockSpec((8, D_HID), lambda i: (0, 0)),
            pl.BlockSpec((D_HID, D_HID), lambda i: (0, 0)),
            pl.BlockSpec((D_HID, D_HID), lambda i: (0, 0)),
        ],
        out_specs=[
            pl.BlockSpec((ROW_BLK, D_HID), lambda i: (i, 0)),
            pl.BlockSpec((ROW_BLK, NCOLS), lambda i: (i, 0)),
        ],
        out_shape=[
            jax.ShapeDtypeStruct((N_NODES, D_HID), jnp.float32),
            jax.ShapeDtypeStruct((N_NODES, NCOLS), jnp.float32),
        ],
    )(p, z1, b1, W2l, W2r)


def _post_body(q_ref, z2c_ref, b_ref, out_ref):
    acc = q_ref[0] + q_ref[1]
    z2c = z2c_ref[...]
    cnt = z2c[:, D_HID:D_HID + 1]
    out_ref[...] = _head(acc, cnt, z2c[:, :D_HID], b_ref[0:1, :])


def _post(q, z2c, b2):
    grid = (N_NODES // ROW_BLK,)
    return pl.pallas_call(
        _post_body,
        grid=grid,
        in_specs=[
            pl.BlockSpec((NC, ROW_BLK, D_HID), lambda i: (0, i, 0)),
            pl.BlockSpec((ROW_BLK, NCOLS), lambda i: (i, 0)),
            pl.BlockSpec((8, D_HID), lambda i: (0, 0)),
        ],
        out_specs=pl.BlockSpec((ROW_BLK, D_HID), lambda i: (i, 0)),
        out_shape=jax.ShapeDtypeStruct((N_NODES, D_HID), jnp.float32),
    )(q, z2c, b2)


def kernel(x, edge_index, W1l, b1, W1r, W2l, b2, W2r):
    src2d = edge_index[0].astype(jnp.int32).reshape(NW * NCHUNK, CHUNK)
    dst2d = edge_index[1].astype(jnp.int32).reshape(NW * NCHUNK, CHUNK)
    zeros48 = jnp.zeros((ROWS_PER_TILE, NCOLS), jnp.float32)
    zeros32 = jnp.zeros((ROWS_PER_TILE, D_HID), jnp.float32)
    b1v = jnp.broadcast_to(b1[None, :], (8, D_HID))
    b2v = jnp.broadcast_to(b2[None, :], (8, D_HID))

    yaug1, z1 = _pre(x, W1l, W1r)
    p = _segsum48(yaug1, src2d, dst2d, zeros48)
    y2, z2c = _mid(p, z1, b1v, W2l, W2r)
    q = _segsum32(y2, src2d, dst2d, zeros32)
    return _post(q, z2c, b2v)


# speedup vs baseline: 1.0696x; 1.0081x over previous
import functools

import jax
import jax.numpy as jnp
from jax import lax
from jax.experimental import pallas as pl
from jax.experimental.pallas import tpu as pltpu
from jax.experimental.pallas import tpu_sc as plsc

N_NODES = 10000
D_IN = 128
D_HID = 32
N_EDGES = 320000

NCOLS = 40
CNT_COL = D_HID
NC, NS = 2, 16
NW = NC * NS
EPW = N_EDGES // NW
CHUNK = 200
NCHUNK = EPW // CHUNK
ROWS_PER_TILE = N_NODES // NS
ACC_ROWS = N_NODES

ROW_BLK = 5000


def _pre_body(x_ref, wl_ref, wr_ref, yaug_ref, z_ref):
    x = x_ref[...]
    y = jnp.dot(x, wl_ref[...], preferred_element_type=jnp.float32)
    z = jnp.dot(x, wr_ref[...], preferred_element_type=jnp.float32)
    r = y.shape[0]
    ones = jnp.ones((r, 1), jnp.float32)
    zeros = jnp.zeros((r, NCOLS - D_HID - 1), jnp.float32)
    yaug_ref[...] = jnp.concatenate([y, ones, zeros], axis=1)
    z_ref[...] = z


def _pre(x, Wl, Wr):
    grid = (N_NODES // ROW_BLK,)
    return pl.pallas_call(
        _pre_body,
        grid=grid,
        in_specs=[
            pl.BlockSpec((ROW_BLK, D_IN), lambda i: (i, 0)),
            pl.BlockSpec((D_IN, D_HID), lambda i: (0, 0)),
            pl.BlockSpec((D_IN, D_HID), lambda i: (0, 0)),
        ],
        out_specs=[
            pl.BlockSpec((ROW_BLK, NCOLS), lambda i: (i, 0)),
            pl.BlockSpec((ROW_BLK, D_HID), lambda i: (i, 0)),
        ],
        out_shape=[
            jax.ShapeDtypeStruct((N_NODES, NCOLS), jnp.float32),
            jax.ShapeDtypeStruct((N_NODES, D_HID), jnp.float32),
        ],
    )(x, Wl, Wr)


NBUF = 10
NGROUP = NCHUNK // NBUF


def _make_segsum(ncols):
    def body(y_hbm, src2d_hbm, dst2d_hbm, zeros_hbm, out_hbm,
             sidx, didx, *bufs_and_sems):
        rows = bufs_and_sems[:NBUF]
        acc = bufs_and_sems[NBUF]
        gsem = bufs_and_sems[NBUF + 1:2 * NBUF + 1]
        ssem = bufs_and_sems[2 * NBUF + 1:]
        cid = lax.axis_index("c")
        sid = lax.axis_index("s")
        wid = sid * NC + cid

        pltpu.sync_copy(zeros_hbm,
                        acc.at[pl.ds(sid * ROWS_PER_TILE, ROWS_PER_TILE)])
        pltpu.sync_copy(src2d_hbm.at[pl.ds(wid * NCHUNK, NCHUNK)], sidx)
        pltpu.sync_copy(dst2d_hbm.at[pl.ds(wid * NCHUNK, NCHUNK)], didx)
        plsc.subcore_barrier()

        def gather(j, b):
            return pltpu.make_async_copy(y_hbm.at[sidx.at[j]], rows[b], gsem[b])

        def scatter_desc(j, b):
            return pltpu.make_async_copy(rows[b], acc.at[didx.at[j]], ssem[b])

        for b in range(NBUF):
            gather(b, b).start()

        def loop_body(k, carry):
            j0 = k * NBUF
            for b in range(NBUF):
                gather(j0 + b, b).wait()
                pltpu.async_copy(rows[b], acc.at[didx.at[j0 + b]], ssem[b],
                                 add=True)
            for b in range(NBUF):
                scatter_desc(j0 + b, b).wait()
                gather(j0 + NBUF + b, b).start()
            return carry

        lax.fori_loop(0, NGROUP - 1, loop_body, 0)
        j0 = (NGROUP - 1) * NBUF
        for b in range(NBUF):
            gather(j0 + b, b).wait()
            pltpu.async_copy(rows[b], acc.at[didx.at[j0 + b]], ssem[b], add=True)
        for b in range(NBUF):
            scatter_desc(j0 + b, b).wait()
        plsc.subcore_barrier()

        pltpu.sync_copy(acc.at[pl.ds(sid * ROWS_PER_TILE, ROWS_PER_TILE)],
                        out_hbm.at[cid, pl.ds(sid * ROWS_PER_TILE, ROWS_PER_TILE)])

    return functools.partial(
        pl.kernel,
        out_type=jax.ShapeDtypeStruct((NC, ACC_ROWS, ncols), jnp.float32),
        mesh=plsc.VectorSubcoreMesh(core_axis_name="c", subcore_axis_name="s"),
        scratch_types=(
            [pltpu.VMEM((NCHUNK, CHUNK), jnp.int32),
             pltpu.VMEM((NCHUNK, CHUNK), jnp.int32)]
            + [pltpu.VMEM((CHUNK, ncols), jnp.float32) for _ in range(NBUF)]
            + [pltpu.VMEM_SHARED((ACC_ROWS, ncols), jnp.float32)]
            + [pltpu.SemaphoreType.DMA for _ in range(2 * NBUF)]
        ),
        compiler_params=pltpu.CompilerParams(use_tc_tiling_on_sc=False),
    )(body)


_segsum48 = _make_segsum(NCOLS)
_segsum32 = _make_segsum(D_HID)


def _head(acc_feat, cnt, z, b):
    mean = acc_feat / jnp.maximum(cnt, 1.0)
    o = mean + b + z
    norm = jnp.sqrt(jnp.sum(o * o, axis=-1, keepdims=True))
    return o / jnp.maximum(norm, 1e-12)


def _mid_body(p_ref, z_ref, b_ref, wl_ref, wr_ref, y2_ref, z2c_ref):
    acc = p_ref[0] + p_ref[1]
    cnt = acc[:, CNT_COL:CNT_COL + 1]
    h = _head(acc[:, :D_HID], cnt, z_ref[...], b_ref[0:1, :])
    h = jnp.maximum(h, 0.0)
    y2 = jnp.dot(h, wl_ref[...], preferred_element_type=jnp.float32)
    z2 = jnp.dot(h, wr_ref[...], preferred_element_type=jnp.float32)
    r = y2.shape[0]
    zeros = jnp.zeros((r, NCOLS - D_HID - 1), jnp.float32)
    y2_ref[...] = y2
    z2c_ref[...] = jnp.concatenate([z2, cnt, zeros], axis=1)


def _mid(p, z1, b1, W2l, W2r):
    grid = (N_NODES // ROW_BLK,)
    return pl.pallas_call(
        _mid_body,
        grid=grid,
        in_specs=[
            pl.BlockSpec((NC, ROW_BLK, NCOLS), lambda i: (0, i, 0)),
            pl.BlockSpec((ROW_BLK, D_HID), lambda i: (i, 0)),
            pl.BlockSpec((8, D_HID), lambda i: (0, 0)),
            pl.BlockSpec((D_HID, D_HID), lambda i: (0, 0)),
            pl.BlockSpec((D_HID, D_HID), lambda i: (0, 0)),
        ],
        out_specs=[
            pl.BlockSpec((ROW_BLK, D_HID), lambda i: (i, 0)),
            pl.BlockSpec((ROW_BLK, NCOLS), lambda i: (i, 0)),
        ],
        out_shape=[
            jax.ShapeDtypeStruct((N_NODES, D_HID), jnp.float32),
            jax.ShapeDtypeStruct((N_NODES, NCOLS), jnp.float32),
        ],
    )(p, z1, b1, W2l, W2r)


def _post_body(q_ref, z2c_ref, b_ref, out_ref):
    acc = q_ref[0] + q_ref[1]
    z2c = z2c_ref[...]
    cnt = z2c[:, D_HID:D_HID + 1]
    out_ref[...] = _head(acc, cnt, z2c[:, :D_HID], b_ref[0:1, :])


def _post(q, z2c, b2):
    grid = (N_NODES // ROW_BLK,)
    return pl.pallas_call(
        _post_body,
        grid=grid,
        in_specs=[
            pl.BlockSpec((NC, ROW_BLK, D_HID), lambda i: (0, i, 0)),
            pl.BlockSpec((ROW_BLK, NCOLS), lambda i: (i, 0)),
            pl.BlockSpec((8, D_HID), lambda i: (0, 0)),
        ],
        out_specs=pl.BlockSpec((ROW_BLK, D_HID), lambda i: (i, 0)),
        out_shape=jax.ShapeDtypeStruct((N_NODES, D_HID), jnp.float32),
    )(q, z2c, b2)


def kernel(x, edge_index, W1l, b1, W1r, W2l, b2, W2r):
    src2d = edge_index[0].astype(jnp.int32).reshape(NW * NCHUNK, CHUNK)
    dst2d = edge_index[1].astype(jnp.int32).reshape(NW * NCHUNK, CHUNK)
    zeros48 = jnp.zeros((ROWS_PER_TILE, NCOLS), jnp.float32)
    zeros32 = jnp.zeros((ROWS_PER_TILE, D_HID), jnp.float32)
    b1v = jnp.broadcast_to(b1[None, :], (8, D_HID))
    b2v = jnp.broadcast_to(b2[None, :], (8, D_HID))

    yaug1, z1 = _pre(x, W1l, W1r)
    p = _segsum48(yaug1, src2d, dst2d, zeros48)
    y2, z2c = _mid(p, z1, b1v, W2l, W2r)
    q = _segsum32(y2, src2d, dst2d, zeros32)
    return _post(q, z2c, b2v)
